# baseline (speedup 1.0000x reference)
import functools

import jax
import jax.numpy as jnp
from jax import lax
from jax.experimental import pallas as pl
from jax.experimental.pallas import tpu as pltpu
from jax.experimental.pallas import tpu_sc as plsc

_B = 16384
_L = 200
_EMB = 32
_SYM = 64
_NCLS = 100

_NW = 32
_RPW = _B // _NW
_CB = 8
_NCHUNK = _RPW // _CB
_IDXROWS = 2 * _CB


@functools.partial(
    pl.kernel,
    out_type=jax.ShapeDtypeStruct((_B, _EMB), jnp.float32),
    mesh=plsc.VectorSubcoreMesh(core_axis_name="c", subcore_axis_name="s"),
    scratch_types=[
        pltpu.VMEM((_IDXROWS, 128), jnp.int32),
        pltpu.VMEM((_IDXROWS, 128), jnp.int32),
        pltpu.VMEM((_CB * _L, _EMB), jnp.float32),
        pltpu.VMEM((_CB * _L, _EMB), jnp.float32),
        pltpu.VMEM((_RPW, _EMB), jnp.float32),
        pltpu.SemaphoreType.DMA,
        pltpu.SemaphoreType.DMA,
        pltpu.SemaphoreType.DMA,
        pltpu.SemaphoreType.DMA,
    ],
    compiler_params=pltpu.CompilerParams(use_tc_tiling_on_sc=False),
)
def _sc_rowsums(ids2_hbm, table_hbm, sums_hbm,
                idx_a, idx_b, rows_a, rows_b, sums_v,
                gsem_a, gsem_b, isem_a, isem_b, ):
    wid = lax.axis_index("s") * 2 + lax.axis_index("c")
    base = wid * _RPW
    idx_bufs = (idx_a, idx_b)
    row_bufs = (rows_a, rows_b)
    gsems = (gsem_a, gsem_b)
    isems = (isem_a, isem_b)

    def gather_parts(slot):
        for r in range(_CB):
            yield (table_hbm.at[idx_bufs[slot].at[2 * r]],
                   row_bufs[slot].at[pl.ds(r * _L, 128)])
            yield (table_hbm.at[idx_bufs[slot].at[2 * r + 1, pl.ds(0, 72)]],
                   row_bufs[slot].at[pl.ds(r * _L + 128, 72)])

    def fire_gathers(slot):
        for src, dst in gather_parts(slot):
            pltpu.async_copy(src, dst, gsems[slot])

    def drain_gathers(slot):
        for src, dst in gather_parts(slot):
            pltpu.make_async_copy(src, dst, gsems[slot]).wait()

    pltpu.sync_copy(ids2_hbm.at[pl.ds(2 * base, _IDXROWS)], idx_bufs[0])
    fire_gathers(0)
    pltpu.async_copy(ids2_hbm.at[pl.ds(2 * (base + _CB), _IDXROWS)],
                     idx_bufs[1], isems[1])

    def outer(io, carry):
        for b in range(2):
            i = io * 2 + b
            slot, nslot = b, 1 - b
            drain_gathers(slot)

            @pl.when(i + 1 < _NCHUNK)
            def _(slot=nslot):
                pltpu.make_async_copy(
                    ids2_hbm.at[pl.ds(0, _IDXROWS)], idx_bufs[slot],
                    isems[slot]).wait()
                fire_gathers(slot)

            @pl.when(i + 2 < _NCHUNK)
            def _(slot=slot, i=i):
                r0 = base + (i + 2) * _CB
                pltpu.async_copy(ids2_hbm.at[pl.ds(2 * r0, _IDXROWS)],
                                 idx_bufs[slot], isems[slot])

            rows_v = row_bufs[slot]
            zero = jnp.zeros((16,), jnp.float32)
            for r in range(_CB):
                rbase = r * _L

                def red_body(j, acc):
                    accs = list(acc)
                    for t in range(4):
                        idx = rbase + j * 4 + t
                        accs[2 * t] = accs[2 * t] + rows_v[idx, pl.ds(0, 16)]
                        accs[2 * t + 1] = (accs[2 * t + 1]
                                           + rows_v[idx, pl.ds(16, 16)])
                    return tuple(accs)

                acc = lax.fori_loop(0, _L // 4, red_body, (zero,) * 8,
                                    unroll=5)
                lo = (acc[0] + acc[2]) + (acc[4] + acc[6])
                hi = (acc[1] + acc[3]) + (acc[5] + acc[7])
                row = i * _CB + r
                sums_v[row, pl.ds(0, 16)] = lo
                sums_v[row, pl.ds(16, 16)] = hi
        return carry

    lax.fori_loop(0, _NCHUNK // 2, outer, 0)
    pltpu.sync_copy(sums_v, sums_hbm.at[pl.ds(base, _RPW)])


_V = 1000000
_TBR = 8000


def _tp_body(t_ref, out_ref):
    x3 = t_ref[...].reshape(_TBR // 4, 4, _EMB)
    for r in range(4):
        out_ref[:, _EMB * r:_EMB * (r + 1)] = x3[:, r, :]


def _tp_call(table):
    return pl.pallas_call(
        _tp_body,
        grid=(_V // _TBR,),
        in_specs=[pl.BlockSpec((_TBR, _EMB), lambda i: (i, 0))],
        out_specs=pl.BlockSpec((_TBR // 4, 128), lambda i: (i, 0)),
        out_shape=jax.ShapeDtypeStruct((_V * _EMB // 128, 128),
                                       jnp.float32),
    )(table)


_BT = 256


def _tc_head(ids_ref, sums_ref, sym_ref, w1_ref, b1_ref, w2_ref, b2_ref,
             wf_ref, bf_ref, out_ref):
    cnt = jnp.sum((ids_ref[...] != 0).astype(jnp.float32), axis=1,
                  keepdims=True)
    mean = sums_ref[...] / jnp.maximum(cnt, 1.0)
    h = jnp.maximum(
        jnp.dot(sym_ref[...], w1_ref[...],
                preferred_element_type=jnp.float32) + b1_ref[...], 0.0)
    h = jnp.maximum(
        jnp.dot(h, w2_ref[...],
                preferred_element_type=jnp.float32) + b2_ref[...], 0.0)
    out_ref[...] = (
        jnp.dot(mean, wf_ref[0:_EMB], preferred_element_type=jnp.float32)
        + jnp.dot(h, wf_ref[_EMB:_EMB + 32],
                  preferred_element_type=jnp.float32)
        + bf_ref[...])


def _tc_call(ids, sums, sym_feats, W1, b1, W2, b2, Wf_p, bf_p):
    grid = (_B // _BT,)
    full = lambda shape: pl.BlockSpec(shape, lambda i: (0, 0))
    return pl.pallas_call(
        _tc_head,
        grid=grid,
        in_specs=[
            pl.BlockSpec((_BT, _L), lambda i: (i, 0)),
            pl.BlockSpec((_BT, _EMB), lambda i: (i, 0)),
            pl.BlockSpec((_BT, _SYM), lambda i: (i, 0)),
            full((_SYM, 32)),
            full((1, 32)),
            full((32, 32)),
            full((1, 32)),
            full((_EMB + 32, 128)),
            full((1, 128)),
        ],
        out_specs=pl.BlockSpec((_BT, 128), lambda i: (i, 0)),
        out_shape=jax.ShapeDtypeStruct((_B, 128), jnp.float32),
    )(ids, sums, sym_feats, W1, b1, W2, b2, Wf_p, bf_p)


def kernel(ids, sym_feats, table, W1, b1, W2, b2, Wf, bf):
    ids = ids.astype(jnp.int32)
    ids2 = jnp.pad(ids, ((0, 0), (0, 56))).reshape(2 * _B, 128)
    sums = _sc_rowsums(ids2, table)

    Wf_p = jnp.zeros((_EMB + 32, 128), jnp.float32).at[:, :_NCLS].set(Wf)
    bf_p = jnp.zeros((1, 128), jnp.float32).at[:, :_NCLS].set(bf)
    out_p = _tc_call(ids, sums, sym_feats, W1, b1.reshape(1, 32), W2,
                     b2.reshape(1, 32), Wf_p, bf_p)
    return out_p[:, :_NCLS]


# speedup vs baseline: 1.0022x; 1.0022x over previous
import functools

import jax
import jax.numpy as jnp
from jax import lax
from jax.experimental import pallas as pl
from jax.experimental.pallas import tpu as pltpu
from jax.experimental.pallas import tpu_sc as plsc

_B = 16384
_L = 200
_EMB = 32
_SYM = 64
_NCLS = 100

_NW = 32
_RPW = _B // _NW
_CB = 8
_NCHUNK = _RPW // _CB
_IDXROWS = 2 * _CB


@functools.partial(
    pl.kernel,
    out_type=jax.ShapeDtypeStruct((_B, _EMB), jnp.float32),
    mesh=plsc.VectorSubcoreMesh(core_axis_name="c", subcore_axis_name="s"),
    scratch_types=[
        pltpu.VMEM((_IDXROWS, 128), jnp.int32),
        pltpu.VMEM((_IDXROWS, 128), jnp.int32),
        pltpu.VMEM((_CB * _L, _EMB), jnp.float32),
        pltpu.VMEM((_CB * _L, _EMB), jnp.float32),
        pltpu.VMEM((_RPW, _EMB), jnp.float32),
        pltpu.SemaphoreType.DMA,
        pltpu.SemaphoreType.DMA,
        pltpu.SemaphoreType.DMA,
        pltpu.SemaphoreType.DMA,
    ],
    compiler_params=pltpu.CompilerParams(use_tc_tiling_on_sc=False),
)
def _sc_rowsums(ids2_hbm, table_hbm, sums_hbm,
                idx_a, idx_b, rows_a, rows_b, sums_v,
                gsem_a, gsem_b, isem_a, isem_b, ):
    wid = lax.axis_index("s") * 2 + lax.axis_index("c")
    base = wid * _RPW
    idx_bufs = (idx_a, idx_b)
    row_bufs = (rows_a, rows_b)
    gsems = (gsem_a, gsem_b)
    isems = (isem_a, isem_b)

    def gather_parts(slot):
        for r in range(_CB):
            yield (table_hbm.at[idx_bufs[slot].at[2 * r]],
                   row_bufs[slot].at[pl.ds(r * _L, 128)])
            yield (table_hbm.at[idx_bufs[slot].at[2 * r + 1, pl.ds(0, 72)]],
                   row_bufs[slot].at[pl.ds(r * _L + 128, 72)])

    def fire_gathers(slot):
        for src, dst in gather_parts(slot):
            pltpu.async_copy(src, dst, gsems[slot])

    def drain_gathers(slot):
        for src, dst in gather_parts(slot):
            pltpu.make_async_copy(src, dst, gsems[slot]).wait()

    pltpu.sync_copy(ids2_hbm.at[pl.ds(2 * base, _IDXROWS)], idx_bufs[0])
    fire_gathers(0)
    pltpu.async_copy(ids2_hbm.at[pl.ds(2 * (base + _CB), _IDXROWS)],
                     idx_bufs[1], isems[1])

    def outer(io, carry):
        for b in range(2):
            i = io * 2 + b
            slot, nslot = b, 1 - b
            drain_gathers(slot)

            @pl.when(i + 1 < _NCHUNK)
            def _(slot=nslot):
                pltpu.make_async_copy(
                    ids2_hbm.at[pl.ds(0, _IDXROWS)], idx_bufs[slot],
                    isems[slot]).wait()
                fire_gathers(slot)

            @pl.when(i + 2 < _NCHUNK)
            def _(slot=slot, i=i):
                r0 = base + (i + 2) * _CB
                pltpu.async_copy(ids2_hbm.at[pl.ds(2 * r0, _IDXROWS)],
                                 idx_bufs[slot], isems[slot])

            rows_v = row_bufs[slot]
            zero = jnp.zeros((16,), jnp.float32)
            for r in range(_CB):
                rbase = r * _L

                def red_body(j, acc):
                    accs = list(acc)
                    for t in range(4):
                        idx = rbase + j * 4 + t
                        accs[2 * t] = accs[2 * t] + rows_v[idx, pl.ds(0, 16)]
                        accs[2 * t + 1] = (accs[2 * t + 1]
                                           + rows_v[idx, pl.ds(16, 16)])
                    return tuple(accs)

                acc = lax.fori_loop(0, _L // 4, red_body, (zero,) * 8,
                                    unroll=2)
                lo = (acc[0] + acc[2]) + (acc[4] + acc[6])
                hi = (acc[1] + acc[3]) + (acc[5] + acc[7])
                row = i * _CB + r
                sums_v[row, pl.ds(0, 16)] = lo
                sums_v[row, pl.ds(16, 16)] = hi
        return carry

    lax.fori_loop(0, _NCHUNK // 2, outer, 0)
    pltpu.sync_copy(sums_v, sums_hbm.at[pl.ds(base, _RPW)])


_V = 1000000
_TBR = 8000


def _tp_body(t_ref, out_ref):
    x3 = t_ref[...].reshape(_TBR // 4, 4, _EMB)
    for r in range(4):
        out_ref[:, _EMB * r:_EMB * (r + 1)] = x3[:, r, :]


def _tp_call(table):
    return pl.pallas_call(
        _tp_body,
        grid=(_V // _TBR,),
        in_specs=[pl.BlockSpec((_TBR, _EMB), lambda i: (i, 0))],
        out_specs=pl.BlockSpec((_TBR // 4, 128), lambda i: (i, 0)),
        out_shape=jax.ShapeDtypeStruct((_V * _EMB // 128, 128),
                                       jnp.float32),
    )(table)


_BT = 256


def _tc_head(ids_ref, sums_ref, sym_ref, w1_ref, b1_ref, w2_ref, b2_ref,
             wf_ref, bf_ref, out_ref):
    cnt = jnp.sum((ids_ref[...] != 0).astype(jnp.float32), axis=1,
                  keepdims=True)
    mean = sums_ref[...] / jnp.maximum(cnt, 1.0)
    h = jnp.maximum(
        jnp.dot(sym_ref[...], w1_ref[...],
                preferred_element_type=jnp.float32) + b1_ref[...], 0.0)
    h = jnp.maximum(
        jnp.dot(h, w2_ref[...],
                preferred_element_type=jnp.float32) + b2_ref[...], 0.0)
    out_ref[...] = (
        jnp.dot(mean, wf_ref[0:_EMB], preferred_element_type=jnp.float32)
        + jnp.dot(h, wf_ref[_EMB:_EMB + 32],
                  preferred_element_type=jnp.float32)
        + bf_ref[...])


def _tc_call(ids, sums, sym_feats, W1, b1, W2, b2, Wf, bf):
    grid = (_B // _BT,)
    full = lambda shape: pl.BlockSpec(shape, lambda i: (0, 0))
    return pl.pallas_call(
        _tc_head,
        grid=grid,
        in_specs=[
            pl.BlockSpec((_BT, _L), lambda i: (i, 0)),
            pl.BlockSpec((_BT, _EMB), lambda i: (i, 0)),
            pl.BlockSpec((_BT, _SYM), lambda i: (i, 0)),
            full((_SYM, 32)),
            full((1, 32)),
            full((32, 32)),
            full((1, 32)),
            full((_EMB + 32, _NCLS)),
            full((1, _NCLS)),
        ],
        out_specs=pl.BlockSpec((_BT, _NCLS), lambda i: (i, 0)),
        out_shape=jax.ShapeDtypeStruct((_B, _NCLS), jnp.float32),
    )(ids, sums, sym_feats, W1, b1, W2, b2, Wf, bf)


def kernel(ids, sym_feats, table, W1, b1, W2, b2, Wf, bf):
    ids = ids.astype(jnp.int32)
    ids2 = jnp.pad(ids, ((0, 0), (0, 56))).reshape(2 * _B, 128)
    sums = _sc_rowsums(ids2, table)

    return _tc_call(ids, sums, sym_feats, W1, b1.reshape(1, 32), W2,
                    b2.reshape(1, 32), Wf, bf.reshape(1, _NCLS))


# speedup vs baseline: 1.0283x; 1.0261x over previous
import functools

import jax
import jax.numpy as jnp
from jax import lax
from jax.experimental import pallas as pl
from jax.experimental.pallas import tpu as pltpu
from jax.experimental.pallas import tpu_sc as plsc

_B = 16384
_L = 200
_EMB = 32
_SYM = 64
_NCLS = 100

_NW = 32
_RPW = _B // _NW
_CB = 8
_NCHUNK = _RPW // _CB
_IDXROWS = 2 * _CB


@functools.partial(
    pl.kernel,
    out_type=jax.ShapeDtypeStruct((_B, _EMB), jnp.float32),
    mesh=plsc.VectorSubcoreMesh(core_axis_name="c", subcore_axis_name="s"),
    scratch_types=[
        pltpu.VMEM((_IDXROWS, 128), jnp.int32),
        pltpu.VMEM((_IDXROWS, 128), jnp.int32),
        pltpu.VMEM((_CB * _L, _EMB), jnp.float32),
        pltpu.VMEM((_CB * _L, _EMB), jnp.float32),
        pltpu.VMEM((_RPW, _EMB), jnp.float32),
        pltpu.SemaphoreType.DMA,
        pltpu.SemaphoreType.DMA,
        pltpu.SemaphoreType.DMA,
        pltpu.SemaphoreType.DMA,
    ],
    compiler_params=pltpu.CompilerParams(use_tc_tiling_on_sc=False),
)
def _sc_rowsums(ids2_hbm, table_hbm, sums_hbm,
                idx_a, idx_b, rows_a, rows_b, sums_v,
                gsem_a, gsem_b, isem_a, isem_b, ):
    wid = lax.axis_index("s") * 2 + lax.axis_index("c")
    base = wid * _RPW
    idx_bufs = (idx_a, idx_b)
    row_bufs = (rows_a, rows_b)
    gsems = (gsem_a, gsem_b)
    isems = (isem_a, isem_b)

    def gather_parts(slot):
        for r in range(_CB):
            yield (table_hbm.at[idx_bufs[slot].at[2 * r]],
                   row_bufs[slot].at[pl.ds(r * _L, 128)])
            yield (table_hbm.at[idx_bufs[slot].at[2 * r + 1, pl.ds(0, 72)]],
                   row_bufs[slot].at[pl.ds(r * _L + 128, 72)])

    def fire_gathers(slot):
        for src, dst in gather_parts(slot):
            pltpu.async_copy(src, dst, gsems[slot])

    def drain_gathers(slot):
        for src, dst in gather_parts(slot):
            pltpu.make_async_copy(src, dst, gsems[slot]).wait()

    pltpu.sync_copy(ids2_hbm.at[pl.ds(2 * base, _IDXROWS)], idx_bufs[0])
    fire_gathers(0)
    pltpu.async_copy(ids2_hbm.at[pl.ds(2 * (base + _CB), _IDXROWS)],
                     idx_bufs[1], isems[1])

    def outer(io, carry):
        for b in range(2):
            i = io * 2 + b
            slot, nslot = b, 1 - b
            drain_gathers(slot)

            @pl.when(i + 1 < _NCHUNK)
            def _(slot=nslot):
                pltpu.make_async_copy(
                    ids2_hbm.at[pl.ds(0, _IDXROWS)], idx_bufs[slot],
                    isems[slot]).wait()
                fire_gathers(slot)

            @pl.when(i + 2 < _NCHUNK)
            def _(slot=slot, i=i):
                r0 = base + (i + 2) * _CB
                pltpu.async_copy(ids2_hbm.at[pl.ds(2 * r0, _IDXROWS)],
                                 idx_bufs[slot], isems[slot])

            rows_v = row_bufs[slot]
            zero = jnp.zeros((16,), jnp.float32)
            for r in range(_CB):
                rbase = r * _L

                def red_body(j, acc):
                    accs = list(acc)
                    for t in range(4):
                        idx = rbase + j * 4 + t
                        accs[2 * t] = accs[2 * t] + rows_v[idx, pl.ds(0, 16)]
                        accs[2 * t + 1] = (accs[2 * t + 1]
                                           + rows_v[idx, pl.ds(16, 16)])
                    return tuple(accs)

                acc = lax.fori_loop(0, _L // 4, red_body, (zero,) * 8,
                                    unroll=2)
                lo = (acc[0] + acc[2]) + (acc[4] + acc[6])
                hi = (acc[1] + acc[3]) + (acc[5] + acc[7])
                row = i * _CB + r
                sums_v[row, pl.ds(0, 16)] = lo
                sums_v[row, pl.ds(16, 16)] = hi
        return carry

    lax.fori_loop(0, _NCHUNK // 2, outer, 0)
    pltpu.sync_copy(sums_v, sums_hbm.at[pl.ds(base, _RPW)])


_V = 1000000
_TBR = 8000


def _tp_body(t_ref, out_ref):
    x3 = t_ref[...].reshape(_TBR // 4, 4, _EMB)
    for r in range(4):
        out_ref[:, _EMB * r:_EMB * (r + 1)] = x3[:, r, :]


def _tp_call(table):
    return pl.pallas_call(
        _tp_body,
        grid=(_V // _TBR,),
        in_specs=[pl.BlockSpec((_TBR, _EMB), lambda i: (i, 0))],
        out_specs=pl.BlockSpec((_TBR // 4, 128), lambda i: (i, 0)),
        out_shape=jax.ShapeDtypeStruct((_V * _EMB // 128, 128),
                                       jnp.float32),
    )(table)


_BT = 512


def _tc_head(ids_ref, sums_ref, sym_ref, w1_ref, b1_ref, w2_ref, b2_ref,
             wf_ref, bf_ref, out_ref):
    cnt = jnp.sum((ids_ref[...] != 0).astype(jnp.float32), axis=1,
                  keepdims=True)
    mean = sums_ref[...] / jnp.maximum(cnt, 1.0)
    h = jnp.maximum(
        jnp.dot(sym_ref[...], w1_ref[...],
                preferred_element_type=jnp.float32) + b1_ref[...], 0.0)
    h = jnp.maximum(
        jnp.dot(h, w2_ref[...],
                preferred_element_type=jnp.float32) + b2_ref[...], 0.0)
    out_ref[...] = (
        jnp.dot(mean, wf_ref[0:_EMB], preferred_element_type=jnp.float32)
        + jnp.dot(h, wf_ref[_EMB:_EMB + 32],
                  preferred_element_type=jnp.float32)
        + bf_ref[...])


def _tc_call(ids, sums, sym_feats, W1, b1, W2, b2, Wf, bf):
    grid = (_B // _BT,)
    full = lambda shape: pl.BlockSpec(shape, lambda i: (0, 0))
    return pl.pallas_call(
        _tc_head,
        grid=grid,
        in_specs=[
            pl.BlockSpec((_BT, _L), lambda i: (i, 0)),
            pl.BlockSpec((_BT, _EMB), lambda i: (i, 0)),
            pl.BlockSpec((_BT, _SYM), lambda i: (i, 0)),
            full((_SYM, 32)),
            full((1, 32)),
            full((32, 32)),
            full((1, 32)),
            full((_EMB + 32, _NCLS)),
            full((1, _NCLS)),
        ],
        out_specs=pl.BlockSpec((_BT, _NCLS), lambda i: (i, 0)),
        out_shape=jax.ShapeDtypeStruct((_B, _NCLS), jnp.float32),
    )(ids, sums, sym_feats, W1, b1, W2, b2, Wf, bf)


def kernel(ids, sym_feats, table, W1, b1, W2, b2, Wf, bf):
    ids = ids.astype(jnp.int32)
    ids2 = jnp.pad(ids, ((0, 0), (0, 56))).reshape(2 * _B, 128)
    sums = _sc_rowsums(ids2, table)

    return _tc_call(ids, sums, sym_feats, W1, b1.reshape(1, 32), W2,
                    b2.reshape(1, 32), Wf, bf.reshape(1, _NCLS))
